# Initial kernel scaffold; baseline (speedup 1.0000x reference)
import jax
import jax.numpy as jnp
from jax.experimental import pallas as pl


def kernel(atomic_numbers, edge_index):
    raise NotImplementedError("write your pallas kernel here")


# speedup vs baseline: 397.6867x; 397.6867x over previous
import functools

import jax
import jax.numpy as jnp
from jax import lax
from jax.experimental import pallas as pl
from jax.experimental.pallas import tpu as pltpu
from jax.experimental.pallas import tpu_sc as plsc

NC = 2
NS = 16
L = 16
NW = NC * NS
N = 100_000
E = 6_400_000
EW = E // NW
CH = 4000
NCH = EW // CH
NPAD = 100_352
ZSL = NPAD // NS

_mesh = plsc.VectorSubcoreMesh(
    core_axis_name="c", subcore_axis_name="s", num_cores=NC, num_subcores=NS)


def _gs_body(f_hbm, gidx_hbm, sidx_hbm, out_hbm,
             table_v, gi_v, si_v, vals_v, zero_v, accum_sh):
    c = lax.axis_index("c")
    s = lax.axis_index("s")
    wid = c * NS + s

    pltpu.sync_copy(f_hbm, table_v)

    def zbody(j, carry):
        zero_v[pl.ds(j * L, L)] = jnp.zeros((L,), jnp.float32)
        return carry

    lax.fori_loop(0, ZSL // L, zbody, 0)
    pltpu.sync_copy(zero_v, accum_sh.at[pl.ds(s * ZSL, ZSL)])
    plsc.subcore_barrier()

    def chunk(i, carry):
        base = wid * EW + i * CH
        pltpu.sync_copy(gidx_hbm.at[pl.ds(base, CH)], gi_v)
        pltpu.sync_copy(sidx_hbm.at[pl.ds(base, CH)], si_v)

        def gbody(j, inner):
            idx = gi_v[pl.ds(j * L, L)]
            vals_v[pl.ds(j * L, L)] = plsc.load_gather(table_v, [idx])
            return inner

        lax.fori_loop(0, CH // L, gbody, 0)
        pltpu.sync_copy(vals_v, accum_sh.at[si_v], add=True)
        return carry

    lax.fori_loop(0, NCH, chunk, 0)
    plsc.subcore_barrier()

    pltpu.sync_copy(accum_sh.at[pl.ds(s * ZSL, ZSL)], zero_v)
    pltpu.sync_copy(zero_v, out_hbm.at[pl.ds(c * NPAD + s * ZSL, ZSL)])


_gs = pl.kernel(
    _gs_body,
    out_type=jax.ShapeDtypeStruct((NC * NPAD,), jnp.float32),
    mesh=_mesh,
    scratch_types=[
        pltpu.VMEM((N,), jnp.float32),
        pltpu.VMEM((CH,), jnp.int32),
        pltpu.VMEM((CH,), jnp.int32),
        pltpu.VMEM((CH,), jnp.float32),
        pltpu.VMEM((ZSL,), jnp.float32),
        pltpu.VMEM_SHARED((NPAD,), jnp.float32),
    ],
    compiler_params=pltpu.CompilerParams(needs_layout_passes=False),
)


def kernel(atomic_numbers, edge_index):
    x0 = atomic_numbers
    src = edge_index[0]
    dst = edge_index[1]
    comb = lambda p: p[:N] + p[NPAD:NPAD + N]
    a = x0 * x0 * x0
    S1 = comb(_gs(a, src, dst))
    x1 = a * S1
    b = x1 * x1 * x1
    S2 = comb(_gs(b, src, dst))
    T2 = comb(_gs(b, dst, src))
    g1 = 3.0 * x1 * x1 * (S2 + T2)
    cval = g1 * a
    U = comb(_gs(cval, dst, src))
    forces = 3.0 * x0 * x0 * (g1 * S1 + U)
    energy = jnp.sum(b * S2)[None]
    return (energy, forces)


# speedup vs baseline: 429.6779x; 1.0804x over previous
import functools

import jax
import jax.numpy as jnp
from jax import lax
from jax.experimental import pallas as pl
from jax.experimental.pallas import tpu as pltpu
from jax.experimental.pallas import tpu_sc as plsc

NC = 2
NS = 16
L = 16
NW = NC * NS
N = 100_000
E = 6_400_000
EW = E // NW
CH = 4000
NCH = EW // CH
CH2 = 2000
NCH2 = EW // CH2
NPAD = 100_352
ZSL = NPAD // NS

_mesh = plsc.VectorSubcoreMesh(
    core_axis_name="c", subcore_axis_name="s", num_cores=NC, num_subcores=NS)


def _gs_body(f_hbm, gidx_hbm, sidx_hbm, out_hbm,
             table_v, gi_v, si_v, vals_v, zero_v, accum_sh):
    c = lax.axis_index("c")
    s = lax.axis_index("s")
    wid = c * NS + s

    pltpu.sync_copy(f_hbm, table_v)

    def zbody(j, carry):
        zero_v[pl.ds(j * L, L)] = jnp.zeros((L,), jnp.float32)
        return carry

    lax.fori_loop(0, ZSL // L, zbody, 0)
    pltpu.sync_copy(zero_v, accum_sh.at[pl.ds(s * ZSL, ZSL)])
    plsc.subcore_barrier()

    def chunk(i, carry):
        base = wid * EW + i * CH
        pltpu.sync_copy(gidx_hbm.at[pl.ds(base, CH)], gi_v)
        pltpu.sync_copy(sidx_hbm.at[pl.ds(base, CH)], si_v)

        def gbody(j, inner):
            idx = gi_v[pl.ds(j * L, L)]
            vals_v[pl.ds(j * L, L)] = plsc.load_gather(table_v, [idx])
            return inner

        lax.fori_loop(0, CH // L, gbody, 0)
        pltpu.sync_copy(vals_v, accum_sh.at[si_v], add=True)
        return carry

    lax.fori_loop(0, NCH, chunk, 0)
    plsc.subcore_barrier()

    pltpu.sync_copy(accum_sh.at[pl.ds(s * ZSL, ZSL)], zero_v)
    pltpu.sync_copy(zero_v, out_hbm.at[pl.ds(c * NPAD + s * ZSL, ZSL)])


def _gs2_body(f_hbm, src_hbm, dst_hbm, outs_hbm, outt_hbm,
              table_v, si_v, di_v, v1_v, v2_v, zero_v, accs_sh, acct_sh):
    c = lax.axis_index("c")
    s = lax.axis_index("s")
    wid = c * NS + s

    pltpu.sync_copy(f_hbm, table_v)

    def zbody(j, carry):
        zero_v[pl.ds(j * L, L)] = jnp.zeros((L,), jnp.float32)
        return carry

    lax.fori_loop(0, ZSL // L, zbody, 0)
    pltpu.sync_copy(zero_v, accs_sh.at[pl.ds(s * ZSL, ZSL)])
    pltpu.sync_copy(zero_v, acct_sh.at[pl.ds(s * ZSL, ZSL)])
    plsc.subcore_barrier()

    def chunk(i, carry):
        base = wid * EW + i * CH2
        pltpu.sync_copy(src_hbm.at[pl.ds(base, CH2)], si_v)
        pltpu.sync_copy(dst_hbm.at[pl.ds(base, CH2)], di_v)

        def gbody(j, inner):
            sidx = si_v[pl.ds(j * L, L)]
            didx = di_v[pl.ds(j * L, L)]
            v1_v[pl.ds(j * L, L)] = plsc.load_gather(table_v, [sidx])
            v2_v[pl.ds(j * L, L)] = plsc.load_gather(table_v, [didx])
            return inner

        lax.fori_loop(0, CH2 // L, gbody, 0)
        pltpu.sync_copy(v1_v, accs_sh.at[di_v], add=True)
        pltpu.sync_copy(v2_v, acct_sh.at[si_v], add=True)
        return carry

    lax.fori_loop(0, NCH2, chunk, 0)
    plsc.subcore_barrier()

    pltpu.sync_copy(accs_sh.at[pl.ds(s * ZSL, ZSL)], zero_v)
    pltpu.sync_copy(zero_v, outs_hbm.at[pl.ds(c * NPAD + s * ZSL, ZSL)])
    pltpu.sync_copy(acct_sh.at[pl.ds(s * ZSL, ZSL)], zero_v)
    pltpu.sync_copy(zero_v, outt_hbm.at[pl.ds(c * NPAD + s * ZSL, ZSL)])


_gs2 = pl.kernel(
    _gs2_body,
    out_type=(jax.ShapeDtypeStruct((NC * NPAD,), jnp.float32),
              jax.ShapeDtypeStruct((NC * NPAD,), jnp.float32)),
    mesh=_mesh,
    scratch_types=[
        pltpu.VMEM((N,), jnp.float32),
        pltpu.VMEM((CH2,), jnp.int32),
        pltpu.VMEM((CH2,), jnp.int32),
        pltpu.VMEM((CH2,), jnp.float32),
        pltpu.VMEM((CH2,), jnp.float32),
        pltpu.VMEM((ZSL,), jnp.float32),
        pltpu.VMEM_SHARED((NPAD,), jnp.float32),
        pltpu.VMEM_SHARED((NPAD,), jnp.float32),
    ],
    compiler_params=pltpu.CompilerParams(needs_layout_passes=False),
)


_gs = pl.kernel(
    _gs_body,
    out_type=jax.ShapeDtypeStruct((NC * NPAD,), jnp.float32),
    mesh=_mesh,
    scratch_types=[
        pltpu.VMEM((N,), jnp.float32),
        pltpu.VMEM((CH,), jnp.int32),
        pltpu.VMEM((CH,), jnp.int32),
        pltpu.VMEM((CH,), jnp.float32),
        pltpu.VMEM((ZSL,), jnp.float32),
        pltpu.VMEM_SHARED((NPAD,), jnp.float32),
    ],
    compiler_params=pltpu.CompilerParams(needs_layout_passes=False),
)


def kernel(atomic_numbers, edge_index):
    x0 = atomic_numbers
    src = edge_index[0]
    dst = edge_index[1]
    comb = lambda p: p[:N] + p[NPAD:NPAD + N]
    a = x0 * x0 * x0
    S1 = comb(_gs(a, src, dst))
    x1 = a * S1
    b = x1 * x1 * x1
    s2p, t2p = _gs2(b, src, dst)
    S2 = comb(s2p)
    T2 = comb(t2p)
    g1 = 3.0 * x1 * x1 * (S2 + T2)
    cval = g1 * a
    U = comb(_gs(cval, dst, src))
    forces = 3.0 * x0 * x0 * (g1 * S1 + U)
    energy = jnp.sum(b * S2)[None]
    return (energy, forces)
